# Initial kernel scaffold; baseline (speedup 1.0000x reference)
import jax
import jax.numpy as jnp
from jax.experimental import pallas as pl


def kernel(feats, coords, mask, z, Wq, Wk, Wv, posW1, posb1, posW2, posb2, Wo, bo):
    raise NotImplementedError("write your pallas kernel here")


# speedup vs baseline: 8.3867x; 8.3867x over previous
import functools
import math

import jax
import jax.numpy as jnp
from jax.experimental import pallas as pl
from jax.experimental.pallas import tpu as pltpu

DIM = 128
L = 64
WIN = 64
STRIDE = 32
N = 16384
SEG = N // L
NW = SEG // STRIDE
INV_TEMP = 1.0 / math.sqrt(DIM)


def _body(f_ref, c_ref, keep_ref, z_ref, wq_ref, wkT_ref, pw2T_ref,
          pb2_ref, w1x_ref, w1y_ref, pb1_ref, wv_ref, woT_ref, bo_ref,
          cx2_ref, cy2_ref, keep2_ref, o_ref, stat_ref):
    i = pl.program_id(0)

    @pl.when(i == 0)
    def _():
        kf = keep2_ref[...]
        n = jnp.float32(N)
        for idx, cref in ((0, cx2_ref), (1, cy2_ref)):
            cm = cref[...] * kf
            s = jnp.sum(cm)
            ss = jnp.sum(cm * cm)
            var = jnp.maximum((ss - s * s / n) / (n - 1.0), 0.0)
            stat_ref[idx] = 1.0 / (jnp.sqrt(var) + 1e-8)

    inv_sx = stat_ref[0]
    inv_sy = stat_ref[1]

    f = f_ref[...]
    mu = jnp.mean(f, axis=1, keepdims=True)
    var = jnp.mean(f * f, axis=1, keepdims=True) - mu * mu
    fn = (f - mu) * jax.lax.rsqrt(jnp.maximum(var, 0.0) + 1e-5)

    keep = keep_ref[...]

    z_row = z_ref[0]
    q_col = jnp.sum(wq_ref[...] * z_row, axis=1, keepdims=True)
    qk_col = jnp.dot(wkT_ref[...], q_col)
    qp_col = jnp.dot(pw2T_ref[...], q_col)
    qb = jnp.sum(q_col * pb2_ref[...])

    a_col = jnp.dot(fn, qk_col) * keep

    c = c_ref[...] * keep
    cx = c[:, 0:1]
    cy = c[:, 1:2]

    w1x = w1x_ref[...]
    w1y = w1y_ref[...]
    pb1 = pb1_ref[...]

    wacc = jnp.zeros((1, DIM), dtype=jnp.float32)
    for w in range(NW):
        s = w * STRIDE
        nw = min(WIN, SEG - s)
        cxw = cx[s:s + nw]
        cyw = cy[s:s + nw]
        cwx = (cxw - jnp.sum(cxw) / nw) * inv_sx
        cwy = (cyw - jnp.sum(cyw) / nw) * inv_sy
        h = jax.nn.relu(cwx * w1x + cwy * w1y + pb1)
        ph = jnp.dot(h, qp_col)
        logits = jnp.clip((a_col[s:s + nw] + ph + qb) * INV_TEMP,
                          -10.0, 10.0)
        m = jnp.max(logits)
        e = jnp.exp(logits - m)
        attn = (e / jnp.sum(e)) * keep[s:s + nw]
        wacc = wacc + jnp.sum(attn * fn[s:s + nw], axis=0, keepdims=True)

    row = wacc * (1.0 / NW)
    zv_col = jnp.sum(wv_ref[...] * row, axis=1, keepdims=True)
    out_row = jnp.sum(woT_ref[...] * zv_col, axis=0, keepdims=True)
    o_ref[0] = out_row + bo_ref[...]


@jax.jit
def kernel(feats, coords, mask, z, Wq, Wk, Wv, posW1, posb1, posW2, posb2,
           Wo, bo):
    keep = 1.0 - mask.astype(jnp.float32)
    keepcol = keep.reshape(N, 1)
    keep2 = keep.reshape(128, 128)
    cx2 = coords[:, 0].reshape(128, 128)
    cy2 = coords[:, 1].reshape(128, 128)

    grid = (L,)
    full = lambda shape: pl.BlockSpec(shape, lambda i: (0, 0))
    seg = lambda shape: pl.BlockSpec(shape, lambda i: (i, 0))

    out = pl.pallas_call(
        _body,
        grid=grid,
        in_specs=[
            seg((SEG, DIM)),
            seg((SEG, 2)),
            seg((SEG, 1)),
            pl.BlockSpec((1, 1, DIM), lambda i: (i, 0, 0)),
            full((DIM, DIM)),
            full((DIM, DIM)),
            full((DIM, DIM)),
            full((DIM, 1)),
            full((1, DIM)),
            full((1, DIM)),
            full((1, DIM)),
            full((DIM, DIM)),
            full((DIM, DIM)),
            full((1, DIM)),
            full((128, 128)),
            full((128, 128)),
            full((128, 128)),
        ],
        out_specs=pl.BlockSpec((1, 1, DIM), lambda i: (i, 0, 0)),
        out_shape=jax.ShapeDtypeStruct((L, 1, DIM), jnp.float32),
        scratch_shapes=[pltpu.SMEM((2,), jnp.float32)],
    )(feats, coords, keepcol, z.reshape(L, 1, DIM), Wq, Wk.T, posW2.T,
      posb2.reshape(DIM, 1),
      posW1[:, 0].reshape(1, DIM), posW1[:, 1].reshape(1, DIM),
      posb1.reshape(1, DIM), Wv, Wo.T, bo.reshape(1, DIM), cx2, cy2, keep2)
    return out.reshape(L, DIM)


# speedup vs baseline: 26.3757x; 3.1449x over previous
import math

import jax
import jax.numpy as jnp
from jax.experimental import pallas as pl
from jax.experimental.pallas import tpu as pltpu

DIM = 128
L = 64
WIN = 64
STRIDE = 32
N = 16384
SEG = N // L
NC = SEG // STRIDE
NW = NC
INV_TEMP = 1.0 / math.sqrt(DIM)
NEG = -1e30


def _body(f_ref, c_ref, keep_ref, z_ref, wqT_ref, wk_ref, pw2_ref,
          pb2_ref, w1x_ref, w1y_ref, pb1_ref, wv_ref, woT_ref, bo_ref,
          cx2_ref, cy2_ref, keep2_ref, o_ref, stat_ref, qk_ref, qp_ref,
          qb_ref):
    i = pl.program_id(0)

    @pl.when(i == 0)
    def _():
        kf = keep2_ref[...]
        n = jnp.float32(N)
        for idx, cref in ((0, cx2_ref), (1, cy2_ref)):
            cm = cref[...] * kf
            s = jnp.sum(cm)
            ss = jnp.sum(cm * cm)
            var = jnp.maximum((ss - s * s / n) / (n - 1.0), 0.0)
            stat_ref[idx] = 1.0 / (jnp.sqrt(var) + 1e-8)
        q_all = jnp.dot(z_ref[...], wqT_ref[...])
        qk_ref[...] = jnp.dot(q_all, wk_ref[...])
        qp_ref[...] = jnp.dot(q_all, pw2_ref[...])
        qb_ref[...] = jnp.dot(q_all, pb2_ref[...])

    inv_sx = stat_ref[0]
    inv_sy = stat_ref[1]

    f = f_ref[...].reshape(NC, STRIDE, DIM)
    mu = jnp.mean(f, axis=2, keepdims=True)
    var = jnp.mean(f * f, axis=2, keepdims=True) - mu * mu
    fn = (f - mu) * jax.lax.rsqrt(jnp.maximum(var, 0.0) + 1e-5)

    keep3 = keep_ref[...].reshape(NC, STRIDE, 1)

    qk_row = qk_ref[pl.ds(i, 1), :].reshape(1, 1, DIM)
    qp_row = qp_ref[pl.ds(i, 1), :].reshape(1, 1, DIM)
    qb = qb_ref[pl.ds(i, 1), :].reshape(1, 1, 1)

    a3 = jnp.sum(fn * qk_row, axis=2, keepdims=True) * keep3

    c = c_ref[...] * keep_ref[...]
    cx3 = c[:, 0:1].reshape(NC, STRIDE, 1)
    cy3 = c[:, 1:2].reshape(NC, STRIDE, 1)

    zero = jnp.zeros((1, 1, 1), jnp.float32)
    row = jax.lax.broadcasted_iota(jnp.int32, (NC, 1, 1), 0)
    cnt = jnp.where(row < NW - 1, float(WIN), float(SEG - (NW - 1) * STRIDE))
    sx = jnp.sum(cx3, axis=1, keepdims=True)
    sy = jnp.sum(cy3, axis=1, keepdims=True)
    mxR = (sx + jnp.concatenate([sx[1:], zero], 0)) / cnt
    myR = (sy + jnp.concatenate([sy[1:], zero], 0)) / cnt
    mxL = jnp.concatenate([zero, mxR[:-1]], 0)
    myL = jnp.concatenate([zero, myR[:-1]], 0)

    w1x = w1x_ref[...].reshape(1, 1, DIM)
    w1y = w1y_ref[...].reshape(1, 1, DIM)
    pb1 = pb1_ref[...].reshape(1, 1, DIM)

    def role_logit(mx, my):
        cwx = (cx3 - mx) * inv_sx
        cwy = (cy3 - my) * inv_sy
        h = jax.nn.relu(cwx * w1x + cwy * w1y + pb1)
        ph = jnp.sum(h * qp_row, axis=2, keepdims=True)
        return jnp.clip((a3 + ph + qb) * INV_TEMP, -10.0, 10.0)

    xR = role_logit(mxR, myR)
    xL = role_logit(mxL, myL)

    xLn = jnp.concatenate([xL[1:], jnp.full((1, STRIDE, 1), NEG)], 0)
    m = jnp.maximum(jnp.max(xR, axis=1, keepdims=True),
                    jnp.max(xLn, axis=1, keepdims=True))
    eR = jnp.exp(xR - m)
    eLn = jnp.exp(xLn - m)
    denom = (jnp.sum(eR, axis=1, keepdims=True)
             + jnp.sum(eLn, axis=1, keepdims=True))
    aR = eR / denom
    aLn = eLn / denom
    aL = jnp.concatenate([jnp.zeros((1, STRIDE, 1), jnp.float32),
                          aLn[:-1]], 0)
    g3 = (aR + aL) * keep3 * (1.0 / NW)

    wrow = jnp.sum(g3 * fn, axis=(0, 1), keepdims=True)[0]
    zv_col = jnp.sum(wv_ref[...] * wrow, axis=1, keepdims=True)
    out_row = jnp.sum(woT_ref[...] * zv_col, axis=0, keepdims=True)
    o_ref[0] = out_row + bo_ref[...]


@jax.jit
def kernel(feats, coords, mask, z, Wq, Wk, Wv, posW1, posb1, posW2, posb2,
           Wo, bo):
    keep = 1.0 - mask.astype(jnp.float32)
    keepcol = keep.reshape(N, 1)
    keep2 = keep.reshape(128, 128)
    cx2 = coords[:, 0].reshape(128, 128)
    cy2 = coords[:, 1].reshape(128, 128)

    grid = (L,)
    full = lambda shape: pl.BlockSpec(shape, lambda i: (0, 0))
    seg = lambda shape: pl.BlockSpec(shape, lambda i: (i, 0))

    out = pl.pallas_call(
        _body,
        grid=grid,
        in_specs=[
            seg((SEG, DIM)),
            seg((SEG, 2)),
            seg((SEG, 1)),
            full((L, DIM)),
            full((DIM, DIM)),
            full((DIM, DIM)),
            full((DIM, DIM)),
            full((DIM, 1)),
            full((1, DIM)),
            full((1, DIM)),
            full((1, DIM)),
            full((DIM, DIM)),
            full((DIM, DIM)),
            full((1, DIM)),
            full((128, 128)),
            full((128, 128)),
            full((128, 128)),
        ],
        out_specs=pl.BlockSpec((1, 1, DIM), lambda i: (i, 0, 0)),
        out_shape=jax.ShapeDtypeStruct((L, 1, DIM), jnp.float32),
        scratch_shapes=[
            pltpu.SMEM((2,), jnp.float32),
            pltpu.VMEM((L, DIM), jnp.float32),
            pltpu.VMEM((L, DIM), jnp.float32),
            pltpu.VMEM((L, 1), jnp.float32),
        ],
    )(feats, coords, keepcol, z, Wq.T, Wk, posW2, posb2.reshape(DIM, 1),
      posW1[:, 0].reshape(1, DIM), posW1[:, 1].reshape(1, DIM),
      posb1.reshape(1, DIM), Wv, Wo.T, bo.reshape(1, DIM), cx2, cy2, keep2)
    return out.reshape(L, DIM)


# speedup vs baseline: 45.2636x; 1.7161x over previous
import math

import jax
import jax.numpy as jnp
from jax.experimental import pallas as pl
from jax.experimental.pallas import tpu as pltpu

DIM = 128
L = 64
WIN = 64
STRIDE = 32
N = 16384
SEG = N // L
SPS = 8
TPS = SPS * SEG
CPS = TPS // STRIDE
NW = SEG // STRIDE
GRID = L // SPS
INV_TEMP = 1.0 / math.sqrt(DIM)
NEG = -1e30


def _body(f_ref, c_ref, keep_ref, z_ref, wqT_ref, wk_ref, pw2_ref,
          pb2_ref, posP_ref, wvT_ref, woT_ref, bo_ref,
          cx2_ref, cy2_ref, keep2_ref, o_ref, stat_ref, qk_ref, qp_ref,
          qb_ref, wacc_ref):
    i = pl.program_id(0)

    @pl.when(i == 0)
    def _():
        kf = keep2_ref[...]
        n = jnp.float32(N)
        for idx, cref in ((0, cx2_ref), (1, cy2_ref)):
            cm = cref[...] * kf
            s = jnp.sum(cm)
            ss = jnp.sum(cm * cm)
            var = jnp.maximum((ss - s * s / n) / (n - 1.0), 0.0)
            stat_ref[idx] = 1.0 / (jnp.sqrt(var) + 1e-8)
        q_all = jnp.dot(z_ref[...], wqT_ref[...])
        qk_ref[...] = jnp.dot(q_all, wk_ref[...])
        qp_ref[...] = jnp.dot(q_all, pw2_ref[...])
        qb_ref[...] = jnp.dot(q_all, pb2_ref[...])

    inv_sx = stat_ref[0]
    inv_sy = stat_ref[1]

    f = f_ref[...]
    ones_col = jnp.ones((DIM, 1), jnp.float32)
    mu = jnp.dot(f, ones_col) * (1.0 / DIM)
    sq = jnp.dot(f * f, ones_col) * (1.0 / DIM)
    rstd = jax.lax.rsqrt(jnp.maximum(sq - mu * mu, 0.0) + 1e-5)
    fn = (f - mu) * rstd

    keep_col = keep_ref[...]

    qk_blk = qk_ref[pl.ds(i * SPS, SPS), :]
    qp_blk = qp_ref[pl.ds(i * SPS, SPS), :]
    qb_blk = qb_ref[pl.ds(i * SPS, SPS), :]
    qb_tok = jnp.broadcast_to(qb_blk.reshape(SPS, 1, 1),
                              (SPS, SEG, 1)).reshape(TPS, 1)

    seg_of_row = jax.lax.broadcasted_iota(jnp.int32, (TPS, SPS), 0) // SEG
    col_id = jax.lax.broadcasted_iota(jnp.int32, (TPS, SPS), 1)
    diag = (seg_of_row == col_id)

    def select(x_all):
        return jnp.sum(jnp.where(diag, x_all, 0.0), axis=1, keepdims=True)

    a_col = select(jax.lax.dot_general(
        fn, qk_blk, (((1,), (1,)), ((), ())))) * keep_col

    c = c_ref[...] * keep_col
    cx = c[:, 0:1]
    cy = c[:, 1:2]

    crow = jax.lax.broadcasted_iota(jnp.int32, (CPS, 1, 1), 0)
    lw = jax.lax.rem(crow, NW)
    is_full = lw < NW - 1
    zero = jnp.zeros((1, 1, 1), jnp.float32)
    sx = jnp.sum(cx.reshape(CPS, STRIDE, 1), axis=1, keepdims=True)
    sy = jnp.sum(cy.reshape(CPS, STRIDE, 1), axis=1, keepdims=True)
    sxn = jnp.concatenate([sx[1:], zero], 0)
    syn = jnp.concatenate([sy[1:], zero], 0)
    last_n = float(SEG - (NW - 1) * STRIDE)
    mxR = jnp.where(is_full, (sx + sxn) / WIN, sx / last_n)
    myR = jnp.where(is_full, (sy + syn) / WIN, sy / last_n)
    mxL = jnp.concatenate([zero, mxR[:-1]], 0)
    myL = jnp.concatenate([zero, myR[:-1]], 0)

    def tok(x3):
        return jnp.broadcast_to(x3, (CPS, STRIDE, 1)).reshape(TPS, 1)

    posP = posP_ref[...]
    ones_tok = jnp.ones((TPS, 1), jnp.float32)

    def role_logit(mx3, my3):
        cw = jnp.concatenate([(cx - tok(mx3)) * inv_sx,
                              (cy - tok(my3)) * inv_sy,
                              ones_tok], axis=1)
        h = jax.nn.relu(jnp.dot(cw, posP))
        ph = select(jax.lax.dot_general(
            h, qp_blk, (((1,), (1,)), ((), ()))))
        return jnp.clip((a_col + ph + qb_tok) * INV_TEMP, -10.0, 10.0)

    xR3 = role_logit(mxR, myR).reshape(CPS, STRIDE, 1)
    xL3 = role_logit(mxL, myL).reshape(CPS, STRIDE, 1)

    xLn = jnp.where(is_full,
                    jnp.concatenate([xL3[1:],
                                     jnp.full((1, STRIDE, 1), NEG)], 0),
                    NEG)
    m = jnp.maximum(jnp.max(xR3, axis=1, keepdims=True),
                    jnp.max(xLn, axis=1, keepdims=True))
    eR = jnp.exp(xR3 - m)
    eLn = jnp.exp(xLn - m)
    denom = (jnp.sum(eR, axis=1, keepdims=True)
             + jnp.sum(eLn, axis=1, keepdims=True))
    aR3 = eR / denom
    aLn3 = eLn / denom
    zeros_chunk = jnp.zeros((1, STRIDE, 1), jnp.float32)
    aL3 = jnp.where(lw == 0, 0.0,
                    jnp.concatenate([zeros_chunk, aLn3[:-1]], 0))
    g_col = ((aR3 + aL3).reshape(TPS, 1)) * keep_col * (1.0 / NW)

    gf = (fn * g_col).reshape(SPS, SEG, DIM)
    wacc_ref[pl.ds(i * SPS, SPS), :] = jnp.sum(gf, axis=1)

    @pl.when(i == GRID - 1)
    def _():
        zv = jnp.dot(wacc_ref[...], wvT_ref[...])
        o_ref[...] = jnp.dot(zv, woT_ref[...]) + bo_ref[...]


@jax.jit
def kernel(feats, coords, mask, z, Wq, Wk, Wv, posW1, posb1, posW2, posb2,
           Wo, bo):
    keep = 1.0 - mask.astype(jnp.float32)
    keepcol = keep.reshape(N, 1)
    keep2 = keep.reshape(128, 128)
    cx2 = coords[:, 0].reshape(128, 128)
    cy2 = coords[:, 1].reshape(128, 128)
    posP = jnp.stack([posW1[:, 0], posW1[:, 1], posb1], axis=0)

    full = lambda shape: pl.BlockSpec(shape, lambda i: (0, 0))
    seg = lambda shape: pl.BlockSpec(shape, lambda i: (i, 0))

    out = pl.pallas_call(
        _body,
        grid=(GRID,),
        in_specs=[
            seg((TPS, DIM)),
            seg((TPS, 2)),
            seg((TPS, 1)),
            full((L, DIM)),
            full((DIM, DIM)),
            full((DIM, DIM)),
            full((DIM, DIM)),
            full((DIM, 1)),
            full((3, DIM)),
            full((DIM, DIM)),
            full((DIM, DIM)),
            full((1, DIM)),
            full((128, 128)),
            full((128, 128)),
            full((128, 128)),
        ],
        out_specs=full((L, DIM)),
        out_shape=jax.ShapeDtypeStruct((L, DIM), jnp.float32),
        scratch_shapes=[
            pltpu.SMEM((2,), jnp.float32),
            pltpu.VMEM((L, DIM), jnp.float32),
            pltpu.VMEM((L, DIM), jnp.float32),
            pltpu.VMEM((L, 1), jnp.float32),
            pltpu.VMEM((L, DIM), jnp.float32),
        ],
    )(feats, coords, keepcol, z, Wq.T, Wk, posW2, posb2.reshape(DIM, 1),
      posP, Wv.T, Wo.T, bo.reshape(1, DIM), cx2, cy2, keep2)
    return out


# speedup vs baseline: 53.3223x; 1.1780x over previous
import math

import jax
import jax.numpy as jnp
from jax.experimental import pallas as pl
from jax.experimental.pallas import tpu as pltpu

DIM = 128
L = 64
WIN = 64
STRIDE = 32
N = 16384
SEG = N // L
SPS = 8
TPS = SPS * SEG
NW = SEG // STRIDE
GRID = L // SPS
INV_TEMP = 1.0 / math.sqrt(DIM)

_DOT_T = (((1,), (1,)), ((), ()))


def _body(f_ref, cxl_ref, cyl_ref, keepl_ref, z_ref,
          wqT_ref, wk_ref, pw2_ref, pb2_ref, posPT_ref, wvT_ref, woT_ref,
          bo_ref, bmR_ref, bmL_ref, bown_ref, bnext_ref, bprev_ref,
          cxf_ref, cyf_ref, keepf_ref, o_ref,
          stat_ref, qk_ref, qp_ref, qb_ref, qs_ref, wacc_ref):
    i = pl.program_id(0)

    @pl.when(i == 0)
    def _():
        kf = keepf_ref[...]
        n = jnp.float32(N)
        for idx, cref in ((0, cxf_ref), (1, cyf_ref)):
            cm = cref[...] * kf
            s = jnp.sum(cm)
            ss = jnp.sum(cm * cm)
            var = jnp.maximum((ss - s * s / n) / (n - 1.0), 0.0)
            stat_ref[idx] = 1.0 / (jnp.sqrt(var) + 1e-8)
        q_all = jnp.dot(z_ref[...], wqT_ref[...])
        qk = jnp.dot(q_all, wk_ref[...])
        qk_ref[...] = qk
        qp_ref[...] = jnp.dot(q_all, pw2_ref[...])
        qb_ref[...] = jnp.dot(q_all, pb2_ref[...])
        qs_ref[...] = jnp.dot(qk, jnp.ones((DIM, 1), jnp.float32))

    inv_sx = stat_ref[0]
    inv_sy = stat_ref[1]

    f = f_ref[...]
    fsq = f * f
    keep_l = keepl_ref[...]

    qk8 = qk_ref[pl.ds(i * SPS, SPS), :]
    qp8 = qp_ref[pl.ds(i * SPS, SPS), :]
    qb8 = qb_ref[pl.ds(i * SPS, SPS), :]
    qs8 = qs_ref[pl.ds(i * SPS, SPS), :]

    ones_row = jnp.ones((1, DIM), jnp.float32)
    fseg = [f[s * SEG:(s + 1) * SEG, :] for s in range(SPS)]
    mu = jnp.concatenate(
        [jax.lax.dot_general(ones_row, fs, _DOT_T) for fs in fseg], 0)
    mu = mu * (1.0 / DIM)
    sq = jnp.concatenate(
        [jax.lax.dot_general(ones_row, fsq[s * SEG:(s + 1) * SEG, :],
                             _DOT_T) for s in range(SPS)], 0) * (1.0 / DIM)
    rstd = jax.lax.rsqrt(jnp.maximum(sq - mu * mu, 0.0) + 1e-5)
    qkf = jnp.concatenate(
        [jax.lax.dot_general(qk8[s:s + 1, :], fseg[s], _DOT_T)
         for s in range(SPS)], 0)
    a = rstd * (qkf - mu * qs8) * keep_l

    cxm = cxl_ref[...] * keep_l
    cym = cyl_ref[...] * keep_l
    mxR = jnp.dot(cxm, bmR_ref[...])
    myR = jnp.dot(cym, bmR_ref[...])
    mxL = jnp.dot(cxm, bmL_ref[...])
    myL = jnp.dot(cym, bmL_ref[...])

    posPT = posPT_ref[...]
    ones_seg = jnp.ones((1, SEG), jnp.float32)

    def role_logit(mx, my):
        cwx = (cxm - mx) * inv_sx
        cwy = (cym - my) * inv_sy
        ph_rows = []
        for s in range(SPS):
            cwt = jnp.concatenate(
                [cwx[s:s + 1, :], cwy[s:s + 1, :], ones_seg], 0)
            hT = jax.nn.relu(jnp.dot(posPT, cwt))
            ph_rows.append(jnp.dot(qp8[s:s + 1, :], hT))
        ph = jnp.concatenate(ph_rows, 0)
        return jnp.clip((a + ph + qb8) * INV_TEMP, -10.0, 10.0)

    eR = jnp.exp(role_logit(mxR, myR))
    eL = jnp.exp(role_logit(mxL, myL))

    den = jnp.dot(eR, bown_ref[...]) + jnp.dot(eL, bnext_ref[...])
    den_prev = jnp.dot(den, bprev_ref[...])
    lane_chunk = jax.lax.broadcasted_iota(jnp.int32, (SPS, SEG), 1) // STRIDE
    aR = eR / den
    aL = jnp.where(lane_chunk == 0, 0.0, eL / den_prev)
    g = (aR + aL) * keep_l * ((1.0 / NW) * rstd)

    w_rows = [jnp.dot(g[s:s + 1, :], fseg[s]) for s in range(SPS)]
    corr = jnp.dot(g * mu, jnp.ones((SEG, 1), jnp.float32))
    wacc_ref[pl.ds(i * SPS, SPS), :] = jnp.concatenate(w_rows, 0) - corr

    @pl.when(i == GRID - 1)
    def _():
        zv = jnp.dot(wacc_ref[...], wvT_ref[...])
        o_ref[...] = jnp.dot(zv, woT_ref[...]) + bo_ref[...]


def _chunk_mats():
    j = jnp.arange(SEG)[:, None] // STRIDE
    k = jnp.arange(SEG)[None, :] // STRIDE
    full = (k < NW - 1)
    last_n = float(SEG - (NW - 1) * STRIDE)
    bmR = jnp.where((j == k) | (full & (j == k + 1)),
                    jnp.where(full, 1.0 / WIN, 1.0 / last_n), 0.0)
    bmL = jnp.where((k >= 1) & ((j == k - 1) | (j == k)), 1.0 / WIN, 0.0)
    bown = (j == k).astype(jnp.float32)
    bnext = (j == k + 1).astype(jnp.float32)
    bprev = jnp.where(j == k - 1, 1.0 / STRIDE, 0.0)
    return bmR.astype(jnp.float32), bmL.astype(jnp.float32), bown, \
        bnext, bprev


@jax.jit
def kernel(feats, coords, mask, z, Wq, Wk, Wv, posW1, posb1, posW2, posb2,
           Wo, bo):
    keep = 1.0 - mask.astype(jnp.float32)
    keepl = keep.reshape(L, SEG)
    cxl = coords[:, 0].reshape(L, SEG)
    cyl = coords[:, 1].reshape(L, SEG)
    posPT = jnp.stack([posW1[:, 0], posW1[:, 1], posb1], axis=1)
    bmR, bmL, bown, bnext, bprev = _chunk_mats()

    full = lambda shape: pl.BlockSpec(shape, lambda i: (0, 0))
    seg = lambda shape: pl.BlockSpec(shape, lambda i: (i, 0))

    out = pl.pallas_call(
        _body,
        grid=(GRID,),
        in_specs=[
            seg((TPS, DIM)),
            seg((SPS, SEG)),
            seg((SPS, SEG)),
            seg((SPS, SEG)),
            full((L, DIM)),
            full((DIM, DIM)),
            full((DIM, DIM)),
            full((DIM, DIM)),
            full((DIM, 1)),
            full((DIM, 3)),
            full((DIM, DIM)),
            full((DIM, DIM)),
            full((1, DIM)),
            full((SEG, SEG)),
            full((SEG, SEG)),
            full((SEG, SEG)),
            full((SEG, SEG)),
            full((SEG, SEG)),
            full((L, SEG)),
            full((L, SEG)),
            full((L, SEG)),
        ],
        out_specs=full((L, DIM)),
        out_shape=jax.ShapeDtypeStruct((L, DIM), jnp.float32),
        scratch_shapes=[
            pltpu.SMEM((2,), jnp.float32),
            pltpu.VMEM((L, DIM), jnp.float32),
            pltpu.VMEM((L, DIM), jnp.float32),
            pltpu.VMEM((L, 1), jnp.float32),
            pltpu.VMEM((L, 1), jnp.float32),
            pltpu.VMEM((L, DIM), jnp.float32),
        ],
    )(feats, cxl, cyl, keepl, z, Wq.T, Wk, posW2,
      posb2.reshape(DIM, 1), posPT, Wv.T, Wo.T, bo.reshape(1, DIM),
      bmR, bmL, bown, bnext, bprev, cxl, cyl, keepl)
    return out
